# Initial kernel scaffold; baseline (speedup 1.0000x reference)
import jax
import jax.numpy as jnp
from jax.experimental import pallas as pl


def kernel(x, edge_attr, go_target, W_edge, W1, b1, W_edge2, Wq, Wk, W_node, W_single, Wp, bp, edge_index, batch_arange, gid):
    raise NotImplementedError("write your pallas kernel here")


# speedup vs baseline: 1.0218x; 1.0218x over previous
import functools

import jax
import jax.numpy as jnp
from jax import lax
from jax.experimental import pallas as pl
from jax.experimental.pallas import tpu as pltpu

N = 10000
E = 320000
D = 128
DE = 16
B = 100
T = 40
NPER = N // B


def _h_body(x_ref, agg_ref, w1_ref, b1_ref, out_ref):
    xa = x_ref[...] + agg_ref[...]
    out_ref[...] = jnp.maximum(xa @ w1_ref[...] + b1_ref[...], 0.0)


def _h_kernel(x, agg, w1, b1):
    blk = 1000
    grid = N // blk
    return pl.pallas_call(
        _h_body,
        grid=(grid,),
        in_specs=[
            pl.BlockSpec((blk, D), lambda i: (i, 0)),
            pl.BlockSpec((blk, D), lambda i: (i, 0)),
            pl.BlockSpec((D, D), lambda i: (0, 0)),
            pl.BlockSpec((1, D), lambda i: (0, 0)),
        ],
        out_specs=pl.BlockSpec((blk, D), lambda i: (i, 0)),
        out_shape=jax.ShapeDtypeStruct((N, D), jnp.float32),
        interpret=False,
    )(x, agg, w1, b1.reshape(1, D))


def kernel(x, edge_attr, go_target, W_edge, W1, b1, W_edge2, Wq, Wk, W_node, W_single, Wp, bp, edge_index, batch_arange, gid):
    n = N
    bsz = B
    nper = NPER
    d = D
    src = edge_index[0]
    dst = edge_index[1]
    e1 = edge_attr @ W_edge
    msg = x[src] + e1
    agg = jax.ops.segment_sum(msg, dst, num_segments=n)
    h = _h_kernel(x, agg, W1, b1)
    e2 = edge_attr @ W_edge2
    a = jax.ops.segment_sum(jax.nn.relu(h[src] + e2), dst, num_segments=n)
    anchor = jax.lax.dynamic_slice_in_dim(h, gid * nper, nper, axis=0)
    q = h @ Wq
    k = anchor @ Wk
    scores = (q @ k.T) / jnp.sqrt(jnp.float32(d))
    alpha = jax.nn.softmax(scores, axis=-1)
    m = alpha @ anchor
    out_multi = jax.nn.relu(jnp.concatenate([h, a, h - m], axis=-1) @ W_node)
    qa = anchor @ Wq
    kall = h @ Wk
    s2 = (qa @ kall.T) / jnp.sqrt(jnp.float32(d))
    s2r = s2.reshape(nper, bsz, nper)
    alpha2 = jax.nn.softmax(s2r, axis=-1)
    hre = h.reshape(bsz, nper, d)
    m2 = jnp.einsum('abn,bnd->abd', alpha2, hre)
    m2 = jnp.transpose(m2, (1, 0, 2))
    anchor_b = jnp.broadcast_to(anchor[None, :, :], (bsz, nper, d))
    out_single = jax.nn.relu(jnp.concatenate([anchor_b, m2], axis=-1) @ W_single)
    batch_ids = batch_arange // nper
    counts = jax.ops.segment_sum(jnp.ones((n,), dtype=jnp.float32), batch_ids, num_segments=bsz)
    g_pool1 = jax.ops.segment_sum(out_multi, batch_ids, num_segments=bsz) / counts[:, None]
    g_pool2 = jnp.mean(out_single, axis=1)
    pred_all1 = g_pool1 @ Wp + bp
    pred_all2 = g_pool2 @ Wp + bp

    def drop_row(arr):
        idx = jnp.arange(bsz - 1, dtype=jnp.int32)
        idx = idx + (idx >= gid).astype(jnp.int32)
        return arr[idx]

    pred1 = drop_row(pred_all1)
    pred2 = drop_row(pred_all2)
    pred = jnp.concatenate([pred1, pred2], axis=-1)
    g_y = go_target[gid]
    by = drop_row(go_target)
    y = jnp.concatenate([by, jnp.broadcast_to(g_y[None, :], by.shape)], axis=-1)
    loss = jnp.mean(jnp.maximum(pred, 0.0) - pred * y + jnp.log1p(jnp.exp(-jnp.abs(pred))))
    return loss


# speedup vs baseline: 1.5026x; 1.4706x over previous
import functools

import jax
import jax.numpy as jnp
from jax import lax
from jax.experimental import pallas as pl
from jax.experimental.pallas import tpu as pltpu
from jax.experimental.pallas import tpu_sc as plsc

N = 10000
E = 320000
D = 128
DE = 16
B = 100
T = 40
NPER = N // B

_INFO = plsc.get_sparse_core_info()
_NC = _INFO.num_cores
_NS = _INFO.num_subcores
_CHUNK = 128
_EPW = E // _NS
_EPWP = 20096
_EP = _NS * _EPWP
_NPAD = 10240
_NROW = _NPAD // _NS
_DH = D // 2


def _sc_pass1(xs, ea_p, src_p, dst_p, zx, ze):
    mesh = plsc.VectorSubcoreMesh(core_axis_name="c", subcore_axis_name="s")

    @functools.partial(
        pl.kernel,
        out_type=[
            jax.ShapeDtypeStruct((_NC, _NPAD, _DH), jnp.float32),
            jax.ShapeDtypeStruct((_NPAD, DE), jnp.float32),
        ],
        mesh=mesh,
        compiler_params=pltpu.CompilerParams(use_tc_tiling_on_sc=False),
        scratch_types=[
            pltpu.VMEM((_CHUNK,), jnp.int32),
            pltpu.VMEM((_CHUNK,), jnp.int32),
            pltpu.VMEM((_CHUNK, _DH), jnp.float32),
            pltpu.VMEM((_CHUNK, DE), jnp.float32),
            pltpu.VMEM_SHARED((_NPAD, _DH), jnp.float32),
            pltpu.VMEM_SHARED((_NPAD, DE), jnp.float32),
            pltpu.SemaphoreType.DMA,
        ],
    )
    def k(xs_hbm, ea_hbm, src_hbm, dst_hbm, zx_hbm, ze_hbm,
          aggx_out, agge_out, idxs_v, idxd_v, rows_v, ea_v, shx, she, sem):
        c = lax.axis_index("c")
        s = lax.axis_index("s")
        r0 = s * _NROW
        pltpu.sync_copy(zx_hbm.at[pl.ds(r0, _NROW)], shx.at[pl.ds(r0, _NROW)])
        pltpu.sync_copy(ze_hbm.at[pl.ds(r0, _NROW)], she.at[pl.ds(r0, _NROW)])
        plsc.subcore_barrier()

        base = s * _EPWP

        def body(it, carry):
            off = base + it * _CHUNK
            pltpu.sync_copy(src_hbm.at[pl.ds(off, _CHUNK)], idxs_v)
            pltpu.sync_copy(dst_hbm.at[pl.ds(off, _CHUNK)], idxd_v)
            pltpu.async_copy(xs_hbm.at[c].at[idxs_v], rows_v, sem).wait()
            pltpu.sync_copy(rows_v, shx.at[idxd_v], add=True)

            @pl.when(c == 0)
            def _():
                pltpu.sync_copy(ea_hbm.at[pl.ds(off, _CHUNK)], ea_v)
                pltpu.sync_copy(ea_v, she.at[idxd_v], add=True)

            return carry

        lax.fori_loop(0, _EPWP // _CHUNK, body, 0)
        plsc.subcore_barrier()
        pltpu.sync_copy(shx.at[pl.ds(r0, _NROW)], aggx_out.at[c, pl.ds(r0, _NROW)])

        @pl.when(c == 0)
        def _():
            pltpu.sync_copy(she.at[pl.ds(r0, _NROW)], agge_out.at[pl.ds(r0, _NROW)])

    return k(xs, ea_p, src_p, dst_p, zx, ze)


def _pad_edges(src, dst, edge_attr):
    pad = _EPWP - _EPW
    src_p = jnp.pad(src.reshape(_NS, _EPW), ((0, 0), (0, pad))).reshape(_EP)
    dst_p = jnp.pad(dst.reshape(_NS, _EPW), ((0, 0), (0, pad)),
                    constant_values=N).reshape(_EP)
    ea_p = jnp.pad(edge_attr.reshape(_NS, _EPW, DE), ((0, 0), (0, pad), (0, 0))
                   ).reshape(_EP, DE)
    return src_p, dst_p, ea_p


def _h_body(x_ref, ax_ref, ae_ref, we_ref, w1_ref, b1_ref, out_ref):
    agg = jnp.concatenate([ax_ref[0], ax_ref[1]], axis=-1) + ae_ref[...] @ we_ref[...]
    out_ref[...] = jnp.maximum((x_ref[...] + agg) @ w1_ref[...] + b1_ref[...], 0.0)


def _h_kernel(x, aggx2, agge, W_edge, W1, b1):
    blk = 1000
    grid = N // blk
    return pl.pallas_call(
        _h_body,
        grid=(grid,),
        in_specs=[
            pl.BlockSpec((blk, D), lambda i: (i, 0)),
            pl.BlockSpec((_NC, blk, _DH), lambda i: (0, i, 0)),
            pl.BlockSpec((blk, DE), lambda i: (i, 0)),
            pl.BlockSpec((DE, D), lambda i: (0, 0)),
            pl.BlockSpec((D, D), lambda i: (0, 0)),
            pl.BlockSpec((1, D), lambda i: (0, 0)),
        ],
        out_specs=pl.BlockSpec((blk, D), lambda i: (i, 0)),
        out_shape=jax.ShapeDtypeStruct((N, D), jnp.float32),
        interpret=False,
    )(x, aggx2, agge, W_edge, W1, b1.reshape(1, D))


def kernel(x, edge_attr, go_target, W_edge, W1, b1, W_edge2, Wq, Wk, W_node, W_single, Wp, bp, edge_index, batch_arange, gid):
    n = N
    bsz = B
    nper = NPER
    d = D
    src = edge_index[0]
    dst = edge_index[1]
    src_p, dst_p, ea_p = _pad_edges(src, dst, edge_attr)
    xs = x.reshape(N, _NC, _DH).transpose(1, 0, 2)
    zx = jnp.zeros((_NPAD, _DH), jnp.float32)
    ze = jnp.zeros((_NPAD, DE), jnp.float32)
    aggx2, agge = _sc_pass1(xs, ea_p, src_p, dst_p, zx, ze)
    h = _h_kernel(x, aggx2, agge, W_edge, W1, b1)
    e2 = edge_attr @ W_edge2
    a = jax.ops.segment_sum(jax.nn.relu(h[src] + e2), dst, num_segments=n)
    anchor = jax.lax.dynamic_slice_in_dim(h, gid * nper, nper, axis=0)
    q = h @ Wq
    k = anchor @ Wk
    scores = (q @ k.T) / jnp.sqrt(jnp.float32(d))
    alpha = jax.nn.softmax(scores, axis=-1)
    m = alpha @ anchor
    out_multi = jax.nn.relu(jnp.concatenate([h, a, h - m], axis=-1) @ W_node)
    qa = anchor @ Wq
    kall = h @ Wk
    s2 = (qa @ kall.T) / jnp.sqrt(jnp.float32(d))
    s2r = s2.reshape(nper, bsz, nper)
    alpha2 = jax.nn.softmax(s2r, axis=-1)
    hre = h.reshape(bsz, nper, d)
    m2 = jnp.einsum('abn,bnd->abd', alpha2, hre)
    m2 = jnp.transpose(m2, (1, 0, 2))
    anchor_b = jnp.broadcast_to(anchor[None, :, :], (bsz, nper, d))
    out_single = jax.nn.relu(jnp.concatenate([anchor_b, m2], axis=-1) @ W_single)
    batch_ids = batch_arange // nper
    counts = jax.ops.segment_sum(jnp.ones((n,), dtype=jnp.float32), batch_ids, num_segments=bsz)
    g_pool1 = jax.ops.segment_sum(out_multi, batch_ids, num_segments=bsz) / counts[:, None]
    g_pool2 = jnp.mean(out_single, axis=1)
    pred_all1 = g_pool1 @ Wp + bp
    pred_all2 = g_pool2 @ Wp + bp

    def drop_row(arr):
        idx = jnp.arange(bsz - 1, dtype=jnp.int32)
        idx = idx + (idx >= gid).astype(jnp.int32)
        return arr[idx]

    pred1 = drop_row(pred_all1)
    pred2 = drop_row(pred_all2)
    pred = jnp.concatenate([pred1, pred2], axis=-1)
    g_y = go_target[gid]
    by = drop_row(go_target)
    y = jnp.concatenate([by, jnp.broadcast_to(g_y[None, :], by.shape)], axis=-1)
    loss = jnp.mean(jnp.maximum(pred, 0.0) - pred * y + jnp.log1p(jnp.exp(-jnp.abs(pred))))
    return loss


# speedup vs baseline: 2.0365x; 1.3553x over previous
import functools

import jax
import jax.numpy as jnp
from jax import lax
from jax.experimental import pallas as pl
from jax.experimental.pallas import tpu as pltpu
from jax.experimental.pallas import tpu_sc as plsc

N = 10000
E = 320000
D = 128
DE = 16
B = 100
T = 40
NPER = N // B

_INFO = plsc.get_sparse_core_info()
_NC = _INFO.num_cores
_NS = _INFO.num_subcores
_CHUNK = 128
_EPW = E // _NS
_EPWP = 20096
_EP = _NS * _EPWP
_NPAD = 10240
_NROW = _NPAD // _NS
_DH = D // 2


def _sc_pass1(xs, ea_p, src_p, dst_p, zx, ze):
    mesh = plsc.VectorSubcoreMesh(core_axis_name="c", subcore_axis_name="s")

    @functools.partial(
        pl.kernel,
        out_type=[
            jax.ShapeDtypeStruct((_NC, _NPAD, _DH), jnp.float32),
            jax.ShapeDtypeStruct((_NPAD, DE), jnp.float32),
        ],
        mesh=mesh,
        compiler_params=pltpu.CompilerParams(use_tc_tiling_on_sc=False),
        scratch_types=[
            pltpu.VMEM((_CHUNK,), jnp.int32),
            pltpu.VMEM((_CHUNK,), jnp.int32),
            pltpu.VMEM((_CHUNK, _DH), jnp.float32),
            pltpu.VMEM((_CHUNK, DE), jnp.float32),
            pltpu.VMEM_SHARED((_NPAD, _DH), jnp.float32),
            pltpu.VMEM_SHARED((_NPAD, DE), jnp.float32),
            pltpu.SemaphoreType.DMA,
        ],
    )
    def k(xs_hbm, ea_hbm, src_hbm, dst_hbm, zx_hbm, ze_hbm,
          aggx_out, agge_out, idxs_v, idxd_v, rows_v, ea_v, shx, she, sem):
        c = lax.axis_index("c")
        s = lax.axis_index("s")
        r0 = s * _NROW
        pltpu.sync_copy(zx_hbm.at[pl.ds(r0, _NROW)], shx.at[pl.ds(r0, _NROW)])
        pltpu.sync_copy(ze_hbm.at[pl.ds(r0, _NROW)], she.at[pl.ds(r0, _NROW)])
        plsc.subcore_barrier()

        base = s * _EPWP

        def body(it, carry):
            off = base + it * _CHUNK
            pltpu.sync_copy(src_hbm.at[pl.ds(off, _CHUNK)], idxs_v)
            pltpu.sync_copy(dst_hbm.at[pl.ds(off, _CHUNK)], idxd_v)
            pltpu.async_copy(xs_hbm.at[c].at[idxs_v], rows_v, sem).wait()
            pltpu.sync_copy(rows_v, shx.at[idxd_v], add=True)

            @pl.when(c == 0)
            def _():
                pltpu.sync_copy(ea_hbm.at[pl.ds(off, _CHUNK)], ea_v)
                pltpu.sync_copy(ea_v, she.at[idxd_v], add=True)

            return carry

        lax.fori_loop(0, _EPWP // _CHUNK, body, 0)
        plsc.subcore_barrier()
        pltpu.sync_copy(shx.at[pl.ds(r0, _NROW)], aggx_out.at[c, pl.ds(r0, _NROW)])

        @pl.when(c == 0)
        def _():
            pltpu.sync_copy(she.at[pl.ds(r0, _NROW)], agge_out.at[pl.ds(r0, _NROW)])

    return k(xs, ea_p, src_p, dst_p, zx, ze)


def _e2_body(ea_ref, w2_ref, out_ref):
    out_ref[0] = ea_ref[...] @ w2_ref[0]


def _e2_kernel(ea_p, W_edge2_s):
    blk = 2048
    return pl.pallas_call(
        _e2_body,
        grid=(_NC, _EP // blk),
        in_specs=[
            pl.BlockSpec((blk, DE), lambda c, i: (i, 0)),
            pl.BlockSpec((1, DE, _DH), lambda c, i: (c, 0, 0)),
        ],
        out_specs=pl.BlockSpec((1, blk, _DH), lambda c, i: (c, i, 0)),
        out_shape=jax.ShapeDtypeStruct((_NC, _EP, _DH), jnp.float32),
        interpret=False,
    )(ea_p, W_edge2_s)


def _sc_pass2(hs, e2s, src_p, dst_p, zx):
    mesh = plsc.VectorSubcoreMesh(core_axis_name="c", subcore_axis_name="s")

    @functools.partial(
        pl.kernel,
        out_type=jax.ShapeDtypeStruct((_NC, _NPAD, _DH), jnp.float32),
        mesh=mesh,
        compiler_params=pltpu.CompilerParams(use_tc_tiling_on_sc=False),
        scratch_types=[
            pltpu.VMEM((_CHUNK,), jnp.int32),
            pltpu.VMEM((_CHUNK,), jnp.int32),
            pltpu.VMEM((_CHUNK, _DH), jnp.float32),
            pltpu.VMEM((_CHUNK, _DH), jnp.float32),
            pltpu.VMEM_SHARED((_NPAD, _DH), jnp.float32),
            pltpu.SemaphoreType.DMA,
        ],
    )
    def k(hs_hbm, e2_hbm, src_hbm, dst_hbm, zx_hbm,
          a_out, idxs_v, idxd_v, hrow_v, e2row_v, sha, sem):
        c = lax.axis_index("c")
        s = lax.axis_index("s")
        r0 = s * _NROW
        pltpu.sync_copy(zx_hbm.at[pl.ds(r0, _NROW)], sha.at[pl.ds(r0, _NROW)])
        plsc.subcore_barrier()

        base = s * _EPWP

        def body(it, carry):
            off = base + it * _CHUNK
            pltpu.sync_copy(src_hbm.at[pl.ds(off, _CHUNK)], idxs_v)
            pltpu.sync_copy(dst_hbm.at[pl.ds(off, _CHUNK)], idxd_v)
            pltpu.async_copy(hs_hbm.at[c].at[idxs_v], hrow_v, sem).wait()
            pltpu.sync_copy(e2_hbm.at[c, pl.ds(off, _CHUNK)], e2row_v)

            def rbody(i, cc):
                for j in range(_DH // 16):
                    sl = pl.ds(j * 16, 16)
                    v = hrow_v[i, sl] + e2row_v[i, sl]
                    hrow_v[i, sl] = jnp.maximum(v, 0.0)
                return cc

            lax.fori_loop(0, _CHUNK, rbody, 0)
            pltpu.sync_copy(hrow_v, sha.at[idxd_v], add=True)
            return carry

        lax.fori_loop(0, _EPWP // _CHUNK, body, 0)
        plsc.subcore_barrier()
        pltpu.sync_copy(sha.at[pl.ds(r0, _NROW)], a_out.at[c, pl.ds(r0, _NROW)])

    return k(hs, e2s, src_p, dst_p, zx)


def _pad_edges(src, dst, edge_attr):
    pad = _EPWP - _EPW
    src_p = jnp.pad(src.reshape(_NS, _EPW), ((0, 0), (0, pad))).reshape(_EP)
    dst_p = jnp.pad(dst.reshape(_NS, _EPW), ((0, 0), (0, pad)),
                    constant_values=N).reshape(_EP)
    ea_p = jnp.pad(edge_attr.reshape(_NS, _EPW, DE), ((0, 0), (0, pad), (0, 0))
                   ).reshape(_EP, DE)
    return src_p, dst_p, ea_p


def _h_body(x_ref, ax_ref, ae_ref, we_ref, w1_ref, b1_ref, out_ref):
    agg = jnp.concatenate([ax_ref[0], ax_ref[1]], axis=-1) + ae_ref[...] @ we_ref[...]
    out_ref[...] = jnp.maximum((x_ref[...] + agg) @ w1_ref[...] + b1_ref[...], 0.0)


def _h_kernel(x, aggx2, agge, W_edge, W1, b1):
    blk = 1000
    grid = N // blk
    return pl.pallas_call(
        _h_body,
        grid=(grid,),
        in_specs=[
            pl.BlockSpec((blk, D), lambda i: (i, 0)),
            pl.BlockSpec((_NC, blk, _DH), lambda i: (0, i, 0)),
            pl.BlockSpec((blk, DE), lambda i: (i, 0)),
            pl.BlockSpec((DE, D), lambda i: (0, 0)),
            pl.BlockSpec((D, D), lambda i: (0, 0)),
            pl.BlockSpec((1, D), lambda i: (0, 0)),
        ],
        out_specs=pl.BlockSpec((blk, D), lambda i: (i, 0)),
        out_shape=jax.ShapeDtypeStruct((N, D), jnp.float32),
        interpret=False,
    )(x, aggx2, agge, W_edge, W1, b1.reshape(1, D))


def kernel(x, edge_attr, go_target, W_edge, W1, b1, W_edge2, Wq, Wk, W_node, W_single, Wp, bp, edge_index, batch_arange, gid):
    n = N
    bsz = B
    nper = NPER
    d = D
    src = edge_index[0]
    dst = edge_index[1]
    src_p, dst_p, ea_p = _pad_edges(src, dst, edge_attr)
    xs = x.reshape(N, _NC, _DH).transpose(1, 0, 2)
    zx = jnp.zeros((_NPAD, _DH), jnp.float32)
    ze = jnp.zeros((_NPAD, DE), jnp.float32)
    aggx2, agge = _sc_pass1(xs, ea_p, src_p, dst_p, zx, ze)
    h = _h_kernel(x, aggx2, agge, W_edge, W1, b1)
    e2s = _e2_kernel(ea_p, W_edge2.reshape(DE, _NC, _DH).transpose(1, 0, 2))
    hs = h.reshape(N, _NC, _DH).transpose(1, 0, 2)
    a2 = _sc_pass2(hs, e2s, src_p, dst_p, zx)
    a = jnp.concatenate([a2[0, :N], a2[1, :N]], axis=-1)
    anchor = jax.lax.dynamic_slice_in_dim(h, gid * nper, nper, axis=0)
    q = h @ Wq
    k = anchor @ Wk
    scores = (q @ k.T) / jnp.sqrt(jnp.float32(d))
    alpha = jax.nn.softmax(scores, axis=-1)
    m = alpha @ anchor
    out_multi = jax.nn.relu(jnp.concatenate([h, a, h - m], axis=-1) @ W_node)
    qa = anchor @ Wq
    kall = h @ Wk
    s2 = (qa @ kall.T) / jnp.sqrt(jnp.float32(d))
    s2r = s2.reshape(nper, bsz, nper)
    alpha2 = jax.nn.softmax(s2r, axis=-1)
    hre = h.reshape(bsz, nper, d)
    m2 = jnp.einsum('abn,bnd->abd', alpha2, hre)
    m2 = jnp.transpose(m2, (1, 0, 2))
    anchor_b = jnp.broadcast_to(anchor[None, :, :], (bsz, nper, d))
    out_single = jax.nn.relu(jnp.concatenate([anchor_b, m2], axis=-1) @ W_single)
    batch_ids = batch_arange // nper
    counts = jax.ops.segment_sum(jnp.ones((n,), dtype=jnp.float32), batch_ids, num_segments=bsz)
    g_pool1 = jax.ops.segment_sum(out_multi, batch_ids, num_segments=bsz) / counts[:, None]
    g_pool2 = jnp.mean(out_single, axis=1)
    pred_all1 = g_pool1 @ Wp + bp
    pred_all2 = g_pool2 @ Wp + bp

    def drop_row(arr):
        idx = jnp.arange(bsz - 1, dtype=jnp.int32)
        idx = idx + (idx >= gid).astype(jnp.int32)
        return arr[idx]

    pred1 = drop_row(pred_all1)
    pred2 = drop_row(pred_all2)
    pred = jnp.concatenate([pred1, pred2], axis=-1)
    g_y = go_target[gid]
    by = drop_row(go_target)
    y = jnp.concatenate([by, jnp.broadcast_to(g_y[None, :], by.shape)], axis=-1)
    loss = jnp.mean(jnp.maximum(pred, 0.0) - pred * y + jnp.log1p(jnp.exp(-jnp.abs(pred))))
    return loss


# speedup vs baseline: 2.3737x; 1.1656x over previous
import functools

import jax
import jax.numpy as jnp
from jax import lax
from jax.experimental import pallas as pl
from jax.experimental.pallas import tpu as pltpu
from jax.experimental.pallas import tpu_sc as plsc

N = 10000
E = 320000
D = 128
DE = 16
B = 100
T = 40
NPER = N // B

_INFO = plsc.get_sparse_core_info()
_NC = _INFO.num_cores
_NS = _INFO.num_subcores
_CHUNK = 128
_NCH = 158
_EPWP = _NCH * _CHUNK
_EP = _NS * _EPWP
_EPW = E // _NS
_NPAD = 10240
_NROW = _NPAD // _NS
_DH = D // 2

_mesh = plsc.VectorSubcoreMesh(core_axis_name="c", subcore_axis_name="s")
_sc_params = pltpu.CompilerParams(use_tc_tiling_on_sc=False)


def _sc_pass1(xs, ea_p, src2, dst2, zx, ze):
    @functools.partial(
        pl.kernel,
        out_type=[
            jax.ShapeDtypeStruct((_NC, _NPAD, _DH), jnp.float32),
            jax.ShapeDtypeStruct((_NC, _NPAD, DE), jnp.float32),
        ],
        mesh=_mesh,
        compiler_params=_sc_params,
        scratch_types=[
            pltpu.VMEM((_NCH, _CHUNK), jnp.int32),
            pltpu.VMEM((_NCH, _CHUNK), jnp.int32),
            pltpu.VMEM((_CHUNK, _DH), jnp.float32),
            pltpu.VMEM((_CHUNK, _DH), jnp.float32),
            pltpu.VMEM((_CHUNK, DE), jnp.float32),
            pltpu.VMEM_SHARED((_NPAD, _DH), jnp.float32),
            pltpu.VMEM_SHARED((_NPAD, DE), jnp.float32),
            pltpu.SemaphoreType.DMA,
            pltpu.SemaphoreType.DMA,
            pltpu.SemaphoreType.DMA,
            pltpu.SemaphoreType.DMA,
        ],
    )
    def k(xs_hbm, ea_hbm, src_hbm, dst_hbm, zx_hbm, ze_hbm,
          aggx_out, agge_out,
          idxs_v, idxd_v, rb0, rb1, ea_v, shx, she, sg0, sg1, ss0, ss1):
        c = lax.axis_index("c")
        s = lax.axis_index("s")
        r0 = s * _NROW
        pltpu.sync_copy(zx_hbm.at[pl.ds(r0, _NROW)], shx.at[pl.ds(r0, _NROW)])
        pltpu.sync_copy(ze_hbm.at[pl.ds(r0, _NROW)], she.at[pl.ds(r0, _NROW)])
        plsc.subcore_barrier()

        pltpu.sync_copy(src_hbm.at[s], idxs_v)
        pltpu.sync_copy(dst_hbm.at[s], idxd_v)
        rbufs = (rb0, rb1)
        sgs = (sg0, sg1)
        sss = (ss0, ss1)

        pltpu.async_copy(xs_hbm.at[c].at[idxs_v.at[0]], rb0, sg0)

        def body(j, cc):
            for b in (0, 1):
                ob = 1 - b
                it = 2 * j + b
                pltpu.make_async_copy(
                    xs_hbm.at[c].at[idxs_v.at[it]], rbufs[b], sgs[b]).wait()
                pltpu.async_copy(rbufs[b], shx.at[idxd_v.at[it]], sss[b], add=True)
                if b == 0:
                    @pl.when(j > 0)
                    def _():
                        pltpu.make_async_copy(
                            rbufs[ob], shx.at[idxd_v.at[it]], sss[ob]).wait()

                    pltpu.async_copy(
                        xs_hbm.at[c].at[idxs_v.at[it + 1]], rbufs[ob], sgs[ob])
                else:
                    pltpu.make_async_copy(
                        rbufs[ob], shx.at[idxd_v.at[it]], sss[ob]).wait()

                    @pl.when(j < _NCH // 2 - 1)
                    def _():
                        pltpu.async_copy(
                            xs_hbm.at[c].at[idxs_v.at[it + 1]], rbufs[ob], sgs[ob])

                @pl.when(c == b)
                def _():
                    off = (s * _NCH + it) * _CHUNK
                    pltpu.sync_copy(ea_hbm.at[pl.ds(off, _CHUNK)], ea_v)
                    pltpu.sync_copy(ea_v, she.at[idxd_v.at[it]], add=True)
            return cc

        lax.fori_loop(0, _NCH // 2, body, 0)
        pltpu.make_async_copy(rb1, shx.at[idxd_v.at[_NCH - 1]], ss1).wait()
        plsc.subcore_barrier()
        pltpu.sync_copy(shx.at[pl.ds(r0, _NROW)], aggx_out.at[c, pl.ds(r0, _NROW)])
        pltpu.sync_copy(she.at[pl.ds(r0, _NROW)], agge_out.at[c, pl.ds(r0, _NROW)])

    return k(xs, ea_p, src2, dst2, zx, ze)


def _sc_pass2(hs, e2s, src2, dst2, zx):
    @functools.partial(
        pl.kernel,
        out_type=jax.ShapeDtypeStruct((_NC, _NPAD, _DH), jnp.float32),
        mesh=_mesh,
        compiler_params=_sc_params,
        scratch_types=[
            pltpu.VMEM((_NCH, _CHUNK), jnp.int32),
            pltpu.VMEM((_NCH, _CHUNK), jnp.int32),
            pltpu.VMEM((_CHUNK, _DH), jnp.float32),
            pltpu.VMEM((_CHUNK, _DH), jnp.float32),
            pltpu.VMEM((_CHUNK, _DH), jnp.float32),
            pltpu.VMEM((_CHUNK, _DH), jnp.float32),
            pltpu.VMEM_SHARED((_NPAD, _DH), jnp.float32),
            pltpu.SemaphoreType.DMA,
            pltpu.SemaphoreType.DMA,
            pltpu.SemaphoreType.DMA,
            pltpu.SemaphoreType.DMA,
            pltpu.SemaphoreType.DMA,
            pltpu.SemaphoreType.DMA,
        ],
    )
    def k(hs_hbm, e2_hbm, src_hbm, dst_hbm, zx_hbm,
          a_out,
          idxs_v, idxd_v, rb0, rb1, e0, e1, sha,
          sg0, sg1, se0, se1, ss0, ss1):
        c = lax.axis_index("c")
        s = lax.axis_index("s")
        r0 = s * _NROW
        pltpu.sync_copy(zx_hbm.at[pl.ds(r0, _NROW)], sha.at[pl.ds(r0, _NROW)])
        plsc.subcore_barrier()

        pltpu.sync_copy(src_hbm.at[s], idxs_v)
        pltpu.sync_copy(dst_hbm.at[s], idxd_v)
        rbufs = (rb0, rb1)
        ebufs = (e0, e1)
        sgs = (sg0, sg1)
        ses = (se0, se1)
        sss = (ss0, ss1)
        ebase = s * _EPWP

        pltpu.async_copy(hs_hbm.at[c].at[idxs_v.at[0]], rb0, sg0)
        pltpu.async_copy(e2_hbm.at[c, pl.ds(ebase, _CHUNK)], e0, se0)

        def body(j, cc):
            for b in (0, 1):
                ob = 1 - b
                it = 2 * j + b
                pltpu.make_async_copy(
                    hs_hbm.at[c].at[idxs_v.at[it]], rbufs[b], sgs[b]).wait()
                pltpu.make_async_copy(
                    e2_hbm.at[c, pl.ds(ebase, _CHUNK)], ebufs[b], ses[b]).wait()

                rb = rbufs[b]
                eb = ebufs[b]

                def rbody(i, cc2):
                    for g in range(_DH // 16):
                        sl = pl.ds(g * 16, 16)
                        rb[i, sl] = jnp.maximum(rb[i, sl] + eb[i, sl], 0.0)
                    return cc2

                lax.fori_loop(0, _CHUNK, rbody, 0)

                pltpu.async_copy(rb, sha.at[idxd_v.at[it]], sss[b], add=True)
                if b == 0:
                    @pl.when(j > 0)
                    def _():
                        pltpu.make_async_copy(
                            rbufs[ob], sha.at[idxd_v.at[it]], sss[ob]).wait()

                    pltpu.async_copy(
                        hs_hbm.at[c].at[idxs_v.at[it + 1]], rbufs[ob], sgs[ob])
                    pltpu.async_copy(
                        e2_hbm.at[c, pl.ds(ebase + (it + 1) * _CHUNK, _CHUNK)],
                        ebufs[ob], ses[ob])
                else:
                    pltpu.make_async_copy(
                        rbufs[ob], sha.at[idxd_v.at[it]], sss[ob]).wait()

                    @pl.when(j < _NCH // 2 - 1)
                    def _():
                        pltpu.async_copy(
                            hs_hbm.at[c].at[idxs_v.at[it + 1]], rbufs[ob], sgs[ob])
                        pltpu.async_copy(
                            e2_hbm.at[c, pl.ds(ebase + (it + 1) * _CHUNK, _CHUNK)],
                            ebufs[ob], ses[ob])
            return cc

        lax.fori_loop(0, _NCH // 2, body, 0)
        pltpu.make_async_copy(rb1, sha.at[idxd_v.at[_NCH - 1]], ss1).wait()
        plsc.subcore_barrier()
        pltpu.sync_copy(sha.at[pl.ds(r0, _NROW)], a_out.at[c, pl.ds(r0, _NROW)])

    return k(hs, e2s, src2, dst2, zx)


def _pad_edges(src, dst, edge_attr):
    pad = _EPWP - _EPW
    src2 = jnp.pad(src.reshape(_NS, _EPW), ((0, 0), (0, pad))
                   ).reshape(_NS, _NCH, _CHUNK)
    dst2 = jnp.pad(dst.reshape(_NS, _EPW), ((0, 0), (0, pad)),
                   constant_values=N).reshape(_NS, _NCH, _CHUNK)
    ea_p = jnp.pad(edge_attr.reshape(_NS, _EPW, DE), ((0, 0), (0, pad), (0, 0))
                   ).reshape(_EP, DE)
    return src2, dst2, ea_p


def _h_body(x_ref, ax_ref, ae_ref, we_ref, w1_ref, b1_ref, out_ref):
    agg = (jnp.concatenate([ax_ref[0], ax_ref[1]], axis=-1)
           + (ae_ref[0] + ae_ref[1]) @ we_ref[...])
    out_ref[...] = jnp.maximum((x_ref[...] + agg) @ w1_ref[...] + b1_ref[...], 0.0)


def _h_kernel(x, aggx2, agge2, W_edge, W1, b1):
    blk = 1000
    grid = N // blk
    return pl.pallas_call(
        _h_body,
        grid=(grid,),
        in_specs=[
            pl.BlockSpec((blk, D), lambda i: (i, 0)),
            pl.BlockSpec((_NC, blk, _DH), lambda i: (0, i, 0)),
            pl.BlockSpec((_NC, blk, DE), lambda i: (0, i, 0)),
            pl.BlockSpec((DE, D), lambda i: (0, 0)),
            pl.BlockSpec((D, D), lambda i: (0, 0)),
            pl.BlockSpec((1, D), lambda i: (0, 0)),
        ],
        out_specs=pl.BlockSpec((blk, D), lambda i: (i, 0)),
        out_shape=jax.ShapeDtypeStruct((N, D), jnp.float32),
        interpret=False,
    )(x, aggx2, agge2, W_edge, W1, b1.reshape(1, D))


def _e2_body(ea_ref, w2_ref, out_ref):
    out_ref[0] = ea_ref[...] @ w2_ref[0]


def _e2_kernel(ea_p, W_edge2_s):
    blk = 2048
    return pl.pallas_call(
        _e2_body,
        grid=(_NC, _EP // blk),
        in_specs=[
            pl.BlockSpec((blk, DE), lambda c, i: (i, 0)),
            pl.BlockSpec((1, DE, _DH), lambda c, i: (c, 0, 0)),
        ],
        out_specs=pl.BlockSpec((1, blk, _DH), lambda c, i: (c, i, 0)),
        out_shape=jax.ShapeDtypeStruct((_NC, _EP, _DH), jnp.float32),
        interpret=False,
    )(ea_p, W_edge2_s)


def kernel(x, edge_attr, go_target, W_edge, W1, b1, W_edge2, Wq, Wk, W_node, W_single, Wp, bp, edge_index, batch_arange, gid):
    n = N
    bsz = B
    nper = NPER
    d = D
    src = edge_index[0]
    dst = edge_index[1]
    src2, dst2, ea_p = _pad_edges(src, dst, edge_attr)
    xs = x.reshape(N, _NC, _DH).transpose(1, 0, 2)
    zx = jnp.zeros((_NPAD, _DH), jnp.float32)
    ze = jnp.zeros((_NPAD, DE), jnp.float32)
    aggx2, agge2 = _sc_pass1(xs, ea_p, src2, dst2, zx, ze)
    h = _h_kernel(x, aggx2, agge2, W_edge, W1, b1)
    e2s = _e2_kernel(ea_p, W_edge2.reshape(DE, _NC, _DH).transpose(1, 0, 2))
    hs = h.reshape(N, _NC, _DH).transpose(1, 0, 2)
    a2 = _sc_pass2(hs, e2s, src2, dst2, zx)
    a = jnp.concatenate([a2[0, :N], a2[1, :N]], axis=-1)
    anchor = jax.lax.dynamic_slice_in_dim(h, gid * nper, nper, axis=0)
    q = h @ Wq
    k = anchor @ Wk
    scores = (q @ k.T) / jnp.sqrt(jnp.float32(d))
    alpha = jax.nn.softmax(scores, axis=-1)
    m = alpha @ anchor
    out_multi = jax.nn.relu(jnp.concatenate([h, a, h - m], axis=-1) @ W_node)
    qa = anchor @ Wq
    kall = h @ Wk
    s2 = (qa @ kall.T) / jnp.sqrt(jnp.float32(d))
    s2r = s2.reshape(nper, bsz, nper)
    alpha2 = jax.nn.softmax(s2r, axis=-1)
    hre = h.reshape(bsz, nper, d)
    m2 = jnp.einsum('abn,bnd->abd', alpha2, hre)
    m2 = jnp.transpose(m2, (1, 0, 2))
    anchor_b = jnp.broadcast_to(anchor[None, :, :], (bsz, nper, d))
    out_single = jax.nn.relu(jnp.concatenate([anchor_b, m2], axis=-1) @ W_single)
    batch_ids = batch_arange // nper
    counts = jax.ops.segment_sum(jnp.ones((n,), dtype=jnp.float32), batch_ids, num_segments=bsz)
    g_pool1 = jax.ops.segment_sum(out_multi, batch_ids, num_segments=bsz) / counts[:, None]
    g_pool2 = jnp.mean(out_single, axis=1)
    pred_all1 = g_pool1 @ Wp + bp
    pred_all2 = g_pool2 @ Wp + bp

    def drop_row(arr):
        idx = jnp.arange(bsz - 1, dtype=jnp.int32)
        idx = idx + (idx >= gid).astype(jnp.int32)
        return arr[idx]

    pred1 = drop_row(pred_all1)
    pred2 = drop_row(pred_all2)
    pred = jnp.concatenate([pred1, pred2], axis=-1)
    g_y = go_target[gid]
    by = drop_row(go_target)
    y = jnp.concatenate([by, jnp.broadcast_to(g_y[None, :], by.shape)], axis=-1)
    loss = jnp.mean(jnp.maximum(pred, 0.0) - pred * y + jnp.log1p(jnp.exp(-jnp.abs(pred))))
    return loss
